# baseline (speedup 1.0000x reference)
import functools

import jax
import jax.numpy as jnp
from jax import lax
from jax.experimental import pallas as pl
from jax.experimental.pallas import tpu as pltpu
from jax.experimental.pallas import tpu_sc as plsc

CH = 128
NW = 32
D = 128
BC = 64
G8 = 8
SPLIT0 = 128


def _make_deg(n_acc, e_pad):
    t_per = e_pad // CH // NW

    mesh = plsc.VectorSubcoreMesh(core_axis_name="c", subcore_axis_name="s")

    @functools.partial(
        pl.kernel,
        mesh=mesh,
        out_type=jax.ShapeDtypeStruct((2, n_acc, D), jnp.float32),
        scratch_types=[
            pltpu.VMEM((1, CH), jnp.int32),
            pltpu.VMEM((CH, D), jnp.float32),
            pltpu.VMEM((BC, D), jnp.float32),
            pltpu.VMEM_SHARED((n_acc, D), jnp.float32),
        ],
    )
    def deg(dst_hbm, out_hbm, dstv, rows, stage, acc):
        c = lax.axis_index("c")
        s = lax.axis_index("s")
        wid = c * 16 + s
        bpt = n_acc // 16 // BC
        r0 = s * bpt * BC
        zeros16 = jnp.zeros((16,), jnp.float32)
        ones16 = jnp.ones((16,), jnp.float32)

        def zrow(r, carry):
            for j in range(D // 16):
                stage[r, pl.ds(j * 16, 16)] = zeros16
            return carry

        lax.fori_loop(0, BC, zrow, 0)
        for k in range(bpt):
            pltpu.sync_copy(stage, acc.at[pl.ds(r0 + k * BC, BC)])

        def orow(r, carry):
            for j in range(D // 16):
                rows[r, pl.ds(j * 16, 16)] = ones16
            return carry

        lax.fori_loop(0, CH, orow, 0)
        plsc.subcore_barrier()

        base = wid * t_per

        def body(t, carry):
            off = (base + t) * CH
            pltpu.sync_copy(dst_hbm.at[pl.ds(off, CH)], dstv.at[0])
            pltpu.sync_copy(rows, acc.at[dstv.at[0]], add=True)
            return carry

        lax.fori_loop(0, t_per, body, 0)
        plsc.subcore_barrier()

        for k in range(bpt):
            pltpu.sync_copy(acc.at[pl.ds(r0 + k * BC, BC)], stage)
            pltpu.sync_copy(stage, out_hbm.at[c, pl.ds(r0 + k * BC, BC)])

    return deg


def _make_agg(n_acc, e_pad):
    n_per_tilepair = e_pad // CH // 16
    t0 = SPLIT0
    t1 = n_per_tilepair - t0
    assert t0 % G8 == 0 and t1 % G8 == 0

    mesh = plsc.VectorSubcoreMesh(core_axis_name="c", subcore_axis_name="s")

    @functools.partial(
        pl.kernel,
        mesh=mesh,
        out_type=jax.ShapeDtypeStruct((2, n_acc, D), jnp.float32),
        scratch_types=[
            pltpu.VMEM((G8, 2, CH), jnp.int32),
            pltpu.VMEM((CH, D), jnp.float32),
            pltpu.VMEM((CH, D), jnp.float32),
            pltpu.VMEM_SHARED((n_acc, D), jnp.float32),
            pltpu.SemaphoreType.DMA,
            pltpu.SemaphoreType.DMA,
        ],
    )
    def agg(x_hbm, sd_hbm, out_hbm, sdv, rows_a, rows_b, acc, sem_a, sem_b):
        c = lax.axis_index("c")
        s = lax.axis_index("s")
        cpb = n_acc // 16 // CH
        r0 = s * cpb * CH
        zeros16 = jnp.zeros((16,), jnp.float32)

        def zrow(r, carry):
            for j in range(D // 16):
                rows_a[r, pl.ds(j * 16, 16)] = zeros16
            return carry

        lax.fori_loop(0, CH, zrow, 0)
        for k in range(cpb):
            pltpu.sync_copy(rows_a, acc.at[pl.ds(r0 + k * CH, CH)])
        plsc.subcore_barrier()

        bufs = ((rows_a, sem_a), (rows_b, sem_b))

        def gat(j):
            rows, sem = bufs[j % 2]
            return pltpu.async_copy(x_hbm.at[sdv.at[j, 0]], rows, sem)

        def scat(j, cp):
            cp.wait()
            rows, _ = bufs[j % 2]
            pltpu.sync_copy(rows, acc.at[sdv.at[j, 1]], add=True)

        def run_edges(base, nt):
            def body(p, carry):
                pltpu.sync_copy(sd_hbm.at[pl.ds(base + p * G8, G8)], sdv)
                cp = gat(0)
                for j in range(G8 - 1):
                    cp_next = gat(j + 1)
                    scat(j, cp)
                    cp = cp_next
                scat(G8 - 1, cp)
                return carry

            lax.fori_loop(0, nt // G8, body, 0)

        lax.cond(c == 0,
                 lambda: run_edges(s * t0, t0),
                 lambda: run_edges(16 * t0 + s * t1, t1))
        plsc.subcore_barrier()

        for k in range(cpb):
            pltpu.sync_copy(acc.at[pl.ds(r0 + k * CH, CH)], rows_a)
            pltpu.sync_copy(rows_a, out_hbm.at[c, pl.ds(r0 + k * CH, CH)])

    return agg


def _dense(acc, degs, xin, wl, wr, b, n, n_acc, br):
    def body(a_ref, d_ref, x_ref, wl_ref, wr_ref, b_ref, o_ref):
        m = (a_ref[0] + a_ref[1]) / jnp.maximum(d_ref[0] + d_ref[1], 1.0)
        o_ref[...] = jnp.tanh(
            jnp.dot(m, wl_ref[...], preferred_element_type=jnp.float32)
            + jnp.dot(x_ref[...], wr_ref[...], preferred_element_type=jnp.float32)
            + b_ref[...])

    grid = n_acc // br
    return pl.pallas_call(
        body,
        grid=(grid,),
        in_specs=[
            pl.BlockSpec((2, br, D), lambda i: (0, i, 0)),
            pl.BlockSpec((2, br, D), lambda i: (0, i, 0)),
            pl.BlockSpec((br, D), lambda i: (i, 0)),
            pl.BlockSpec((D, D), lambda i: (0, 0)),
            pl.BlockSpec((D, D), lambda i: (0, 0)),
            pl.BlockSpec((1, D), lambda i: (0, 0)),
        ],
        out_specs=pl.BlockSpec((br, D), lambda i: (i, 0)),
        out_shape=jax.ShapeDtypeStruct((n, D), jnp.float32),
    )(acc, degs, xin, wl, wr, b)


def kernel(x, edge_index, W1l, W1r, b1, W2l, W2r, b2):
    n = x.shape[0]
    e = edge_index.shape[1]

    n_acc = ((n + 1 + 16 * CH - 1) // (16 * CH)) * (16 * CH)
    gran = 16 * G8 * CH
    e_pad = ((e + 2 * gran - 1) // (2 * gran)) * (2 * gran)

    ei = edge_index.astype(jnp.int32)
    pad = e_pad - e
    src = jnp.concatenate([ei[0], jnp.zeros((pad,), jnp.int32)])
    dst = jnp.concatenate([ei[1], jnp.full((pad,), n, jnp.int32)])
    sd = jnp.stack([src.reshape(-1, CH), dst.reshape(-1, CH)], axis=1)

    deg_k = _make_deg(n_acc, e_pad)
    agg = _make_agg(n_acc, e_pad)
    b1r = b1.reshape(1, D)
    b2r = b2.reshape(1, D)

    degs = deg_k(dst)
    acc1 = agg(x, sd)
    h = _dense(acc1, degs, x, W1l, W1r, b1r, n, n_acc, 1024)
    acc2 = agg(h, sd)
    out = _dense(acc2, degs, h, W2l, W2r, b2r, n, n_acc, 1024)
    return out


# speedup vs baseline: 1.0032x; 1.0032x over previous
import functools

import jax
import jax.numpy as jnp
from jax import lax
from jax.experimental import pallas as pl
from jax.experimental.pallas import tpu as pltpu
from jax.experimental.pallas import tpu_sc as plsc

CH = 128
NW = 32
D = 128
BC = 64
G8 = 4
SPLIT0 = 128


def _make_deg(n_acc, e_pad):
    t_per = e_pad // CH // NW

    mesh = plsc.VectorSubcoreMesh(core_axis_name="c", subcore_axis_name="s")

    @functools.partial(
        pl.kernel,
        mesh=mesh,
        out_type=jax.ShapeDtypeStruct((2, n_acc, D), jnp.float32),
        scratch_types=[
            pltpu.VMEM((1, CH), jnp.int32),
            pltpu.VMEM((CH, D), jnp.float32),
            pltpu.VMEM((BC, D), jnp.float32),
            pltpu.VMEM_SHARED((n_acc, D), jnp.float32),
        ],
    )
    def deg(dst_hbm, out_hbm, dstv, rows, stage, acc):
        c = lax.axis_index("c")
        s = lax.axis_index("s")
        wid = c * 16 + s
        bpt = n_acc // 16 // BC
        r0 = s * bpt * BC
        zeros16 = jnp.zeros((16,), jnp.float32)
        ones16 = jnp.ones((16,), jnp.float32)

        def zrow(r, carry):
            for j in range(D // 16):
                stage[r, pl.ds(j * 16, 16)] = zeros16
            return carry

        lax.fori_loop(0, BC, zrow, 0)
        for k in range(bpt):
            pltpu.sync_copy(stage, acc.at[pl.ds(r0 + k * BC, BC)])

        def orow(r, carry):
            for j in range(D // 16):
                rows[r, pl.ds(j * 16, 16)] = ones16
            return carry

        lax.fori_loop(0, CH, orow, 0)
        plsc.subcore_barrier()

        base = wid * t_per

        def body(t, carry):
            off = (base + t) * CH
            pltpu.sync_copy(dst_hbm.at[pl.ds(off, CH)], dstv.at[0])
            pltpu.sync_copy(rows, acc.at[dstv.at[0]], add=True)
            return carry

        lax.fori_loop(0, t_per, body, 0)
        plsc.subcore_barrier()

        for k in range(bpt):
            pltpu.sync_copy(acc.at[pl.ds(r0 + k * BC, BC)], stage)
            pltpu.sync_copy(stage, out_hbm.at[c, pl.ds(r0 + k * BC, BC)])

    return deg


def _make_agg(n_acc, e_pad):
    n_per_tilepair = e_pad // CH // 16
    t0 = SPLIT0
    t1 = n_per_tilepair - t0
    assert t0 % G8 == 0 and t1 % G8 == 0

    mesh = plsc.VectorSubcoreMesh(core_axis_name="c", subcore_axis_name="s")

    @functools.partial(
        pl.kernel,
        mesh=mesh,
        out_type=jax.ShapeDtypeStruct((2, n_acc, D), jnp.float32),
        scratch_types=[
            pltpu.VMEM((G8, 2, CH), jnp.int32),
            pltpu.VMEM((CH, D), jnp.float32),
            pltpu.VMEM((CH, D), jnp.float32),
            pltpu.VMEM_SHARED((n_acc, D), jnp.float32),
            pltpu.SemaphoreType.DMA,
            pltpu.SemaphoreType.DMA,
        ],
    )
    def agg(x_hbm, sd_hbm, out_hbm, sdv, rows_a, rows_b, acc, sem_a, sem_b):
        c = lax.axis_index("c")
        s = lax.axis_index("s")
        cpb = n_acc // 16 // CH
        r0 = s * cpb * CH
        zeros16 = jnp.zeros((16,), jnp.float32)

        def zrow(r, carry):
            for j in range(D // 16):
                rows_a[r, pl.ds(j * 16, 16)] = zeros16
            return carry

        lax.fori_loop(0, CH, zrow, 0)
        for k in range(cpb):
            pltpu.sync_copy(rows_a, acc.at[pl.ds(r0 + k * CH, CH)])
        plsc.subcore_barrier()

        bufs = ((rows_a, sem_a), (rows_b, sem_b))

        def gat(j):
            rows, sem = bufs[j % 2]
            return pltpu.async_copy(x_hbm.at[sdv.at[j, 0]], rows, sem)

        def scat(j, cp):
            cp.wait()
            rows, _ = bufs[j % 2]
            pltpu.sync_copy(rows, acc.at[sdv.at[j, 1]], add=True)

        def run_edges(base, nt):
            def body(p, carry):
                pltpu.sync_copy(sd_hbm.at[pl.ds(base + p * G8, G8)], sdv)
                cp = gat(0)
                for j in range(G8 - 1):
                    cp_next = gat(j + 1)
                    scat(j, cp)
                    cp = cp_next
                scat(G8 - 1, cp)
                return carry

            lax.fori_loop(0, nt // G8, body, 0)

        lax.cond(c == 0,
                 lambda: run_edges(s * t0, t0),
                 lambda: run_edges(16 * t0 + s * t1, t1))
        plsc.subcore_barrier()

        for k in range(cpb):
            pltpu.sync_copy(acc.at[pl.ds(r0 + k * CH, CH)], rows_a)
            pltpu.sync_copy(rows_a, out_hbm.at[c, pl.ds(r0 + k * CH, CH)])

    return agg


def _dense(acc, degs, xin, wl, wr, b, n, n_acc, br):
    def body(a_ref, d_ref, x_ref, wl_ref, wr_ref, b_ref, o_ref):
        m = (a_ref[0] + a_ref[1]) / jnp.maximum(d_ref[0] + d_ref[1], 1.0)
        o_ref[...] = jnp.tanh(
            jnp.dot(m, wl_ref[...], preferred_element_type=jnp.float32)
            + jnp.dot(x_ref[...], wr_ref[...], preferred_element_type=jnp.float32)
            + b_ref[...])

    grid = n_acc // br
    return pl.pallas_call(
        body,
        grid=(grid,),
        in_specs=[
            pl.BlockSpec((2, br, D), lambda i: (0, i, 0)),
            pl.BlockSpec((2, br, D), lambda i: (0, i, 0)),
            pl.BlockSpec((br, D), lambda i: (i, 0)),
            pl.BlockSpec((D, D), lambda i: (0, 0)),
            pl.BlockSpec((D, D), lambda i: (0, 0)),
            pl.BlockSpec((1, D), lambda i: (0, 0)),
        ],
        out_specs=pl.BlockSpec((br, D), lambda i: (i, 0)),
        out_shape=jax.ShapeDtypeStruct((n, D), jnp.float32),
    )(acc, degs, xin, wl, wr, b)


def kernel(x, edge_index, W1l, W1r, b1, W2l, W2r, b2):
    n = x.shape[0]
    e = edge_index.shape[1]

    n_acc = ((n + 1 + 16 * CH - 1) // (16 * CH)) * (16 * CH)
    gran = 16 * G8 * CH
    e_pad = ((e + 2 * gran - 1) // (2 * gran)) * (2 * gran)

    ei = edge_index.astype(jnp.int32)
    pad = e_pad - e
    src = jnp.concatenate([ei[0], jnp.zeros((pad,), jnp.int32)])
    dst = jnp.concatenate([ei[1], jnp.full((pad,), n, jnp.int32)])
    sd = jnp.stack([src.reshape(-1, CH), dst.reshape(-1, CH)], axis=1)

    deg_k = _make_deg(n_acc, e_pad)
    agg = _make_agg(n_acc, e_pad)
    b1r = b1.reshape(1, D)
    b2r = b2.reshape(1, D)

    degs = deg_k(dst)
    acc1 = agg(x, sd)
    h = _dense(acc1, degs, x, W1l, W1r, b1r, n, n_acc, 1024)
    acc2 = agg(h, sd)
    out = _dense(acc2, degs, h, W2l, W2r, b2r, n, n_acc, 1024)
    return out


# speedup vs baseline: 1.7337x; 1.7281x over previous
import functools

import jax
import jax.numpy as jnp
from jax import lax
from jax.experimental import pallas as pl
from jax.experimental.pallas import tpu as pltpu
from jax.experimental.pallas import tpu_sc as plsc

CH = 128
NW = 32
D = 128
BC = 64
SPLIT0 = 131


def _make_deg(n_acc, e_pad):
    t_per = e_pad // CH // NW

    mesh = plsc.VectorSubcoreMesh(core_axis_name="c", subcore_axis_name="s")

    @functools.partial(
        pl.kernel,
        mesh=mesh,
        out_type=jax.ShapeDtypeStruct((2, n_acc, D), jnp.float32),
        scratch_types=[
            pltpu.VMEM((1, CH), jnp.int32),
            pltpu.VMEM((CH, D), jnp.float32),
            pltpu.VMEM((BC, D), jnp.float32),
            pltpu.VMEM_SHARED((n_acc, D), jnp.float32),
        ],
    )
    def deg(dst_hbm, out_hbm, dstv, rows, stage, acc):
        c = lax.axis_index("c")
        s = lax.axis_index("s")
        wid = c * 16 + s
        bpt = n_acc // 16 // BC
        r0 = s * bpt * BC
        zeros16 = jnp.zeros((16,), jnp.float32)
        ones16 = jnp.ones((16,), jnp.float32)

        def zrow(r, carry):
            for j in range(D // 16):
                stage[r, pl.ds(j * 16, 16)] = zeros16
            return carry

        lax.fori_loop(0, BC, zrow, 0)
        for k in range(bpt):
            pltpu.sync_copy(stage, acc.at[pl.ds(r0 + k * BC, BC)])

        def orow(r, carry):
            for j in range(D // 16):
                rows[r, pl.ds(j * 16, 16)] = ones16
            return carry

        lax.fori_loop(0, CH, orow, 0)
        plsc.subcore_barrier()

        base = wid * t_per

        def body(t, carry):
            off = (base + t) * CH
            pltpu.sync_copy(dst_hbm.at[pl.ds(off, CH)], dstv.at[0])
            pltpu.sync_copy(rows, acc.at[dstv.at[0]], add=True)
            return carry

        lax.fori_loop(0, t_per, body, 0)
        plsc.subcore_barrier()

        for k in range(bpt):
            pltpu.sync_copy(acc.at[pl.ds(r0 + k * BC, BC)], stage)
            pltpu.sync_copy(stage, out_hbm.at[c, pl.ds(r0 + k * BC, BC)])

    return deg


def _make_agg(n_acc, e_pad):
    n_per_tilepair = e_pad // CH // 16
    t0 = SPLIT0
    t1 = n_per_tilepair - t0

    mesh = plsc.VectorSubcoreMesh(core_axis_name="c", subcore_axis_name="s")

    @functools.partial(
        pl.kernel,
        mesh=mesh,
        out_type=jax.ShapeDtypeStruct((2, n_acc, D), jnp.float32),
        scratch_types=[
            pltpu.VMEM((2, 2, CH), jnp.int32),
            pltpu.VMEM((CH, D), jnp.float32),
            pltpu.VMEM((CH, D), jnp.float32),
            pltpu.VMEM_SHARED((n_acc, D), jnp.float32),
            pltpu.SemaphoreType.DMA,
            pltpu.SemaphoreType.DMA,
        ],
    )
    def agg(x_hbm, sd_hbm, out_hbm, sdv, rows_a, rows_b, acc, sem_a, sem_b):
        c = lax.axis_index("c")
        s = lax.axis_index("s")
        cpb = n_acc // 16 // CH
        r0 = s * cpb * CH
        zeros16 = jnp.zeros((16,), jnp.float32)

        def zrow(r, carry):
            for j in range(D // 16):
                rows_a[r, pl.ds(j * 16, 16)] = zeros16
            return carry

        lax.fori_loop(0, CH, zrow, 0)
        for k in range(cpb):
            pltpu.sync_copy(rows_a, acc.at[pl.ds(r0 + k * CH, CH)])
        plsc.subcore_barrier()

        def load_start(t, sdbuf, rows, sem):
            pltpu.sync_copy(sd_hbm.at[t], sdbuf)
            return pltpu.async_copy(x_hbm.at[sdbuf.at[0]], rows, sem)

        def scat(sdbuf, rows, cp):
            cp.wait()
            pltpu.sync_copy(rows, acc.at[sdbuf.at[1]], add=True)

        def run_edges(base, nt):
            cp_a = load_start(base, sdv.at[0], rows_a, sem_a)

            def body(g, carry):
                t = base + 2 * g
                cp_b = load_start(t + 1, sdv.at[1], rows_b, sem_b)
                scat(sdv.at[0], rows_a, cp_a)
                load_start(t + 2, sdv.at[0], rows_a, sem_a)
                scat(sdv.at[1], rows_b, cp_b)
                return carry

            lax.fori_loop(0, (nt - 1) // 2, body, 0)
            scat(sdv.at[0], rows_a, cp_a)

        lax.cond(c == 0,
                 lambda: run_edges(s * t0, t0),
                 lambda: run_edges(16 * t0 + s * t1, t1))
        plsc.subcore_barrier()

        for k in range(cpb):
            pltpu.sync_copy(acc.at[pl.ds(r0 + k * CH, CH)], rows_a)
            pltpu.sync_copy(rows_a, out_hbm.at[c, pl.ds(r0 + k * CH, CH)])

    return agg


def _dense(acc, degs, xin, wl, wr, b, n, n_acc, br):
    def body(a_ref, d_ref, x_ref, wl_ref, wr_ref, b_ref, o_ref):
        m = (a_ref[0] + a_ref[1]) / jnp.maximum(d_ref[0] + d_ref[1], 1.0)
        o_ref[...] = jnp.tanh(
            jnp.dot(m, wl_ref[...], preferred_element_type=jnp.float32)
            + jnp.dot(x_ref[...], wr_ref[...], preferred_element_type=jnp.float32)
            + b_ref[...])

    grid = n_acc // br
    return pl.pallas_call(
        body,
        grid=(grid,),
        in_specs=[
            pl.BlockSpec((2, br, D), lambda i: (0, i, 0)),
            pl.BlockSpec((2, br, D), lambda i: (0, i, 0)),
            pl.BlockSpec((br, D), lambda i: (i, 0)),
            pl.BlockSpec((D, D), lambda i: (0, 0)),
            pl.BlockSpec((D, D), lambda i: (0, 0)),
            pl.BlockSpec((1, D), lambda i: (0, 0)),
        ],
        out_specs=pl.BlockSpec((br, D), lambda i: (i, 0)),
        out_shape=jax.ShapeDtypeStruct((n, D), jnp.float32),
    )(acc, degs, xin, wl, wr, b)


def kernel(x, edge_index, W1l, W1r, b1, W2l, W2r, b2):
    n = x.shape[0]
    e = edge_index.shape[1]

    n_acc = ((n + 1 + 16 * CH - 1) // (16 * CH)) * (16 * CH)
    e_pad = ((e + NW * CH - 1) // (NW * CH)) * (NW * CH)

    ei = edge_index.astype(jnp.int32)
    pad = e_pad - e
    src = jnp.concatenate([ei[0], jnp.zeros((pad,), jnp.int32)])
    dst = jnp.concatenate([ei[1], jnp.full((pad,), n, jnp.int32)])
    sd = jnp.stack([src.reshape(-1, CH), dst.reshape(-1, CH)], axis=1)

    deg_k = _make_deg(n_acc, e_pad)
    agg = _make_agg(n_acc, e_pad)
    b1r = b1.reshape(1, D)
    b2r = b2.reshape(1, D)

    degs = deg_k(dst)
    acc1 = agg(x, sd)
    h = _dense(acc1, degs, x, W1l, W1r, b1r, n, n_acc, 1024)
    acc2 = agg(h, sd)
    out = _dense(acc2, degs, h, W2l, W2r, b2r, n, n_acc, 1024)
    return out
